# baseline (speedup 1.0000x reference)
import functools

import jax
import jax.numpy as jnp
from jax import lax
from jax.experimental import pallas as pl
from jax.experimental.pallas import tpu as pltpu
from jax.experimental.pallas import tpu_sc as plsc

D = 64
SCALE = 8.0
NC, NS = 2, 16
NW = NC * NS
IDXW = 128
SUB = 4
CHUNK = IDXW * SUB
B = 4096 * 200
LANES = 16


def _emb_body(x_hbm, table_hbm, out_hbm, idx_v, rows_v, gsem):
    wid = lax.axis_index("s") * NC + lax.axis_index("c")
    chunks_per_w = B // (NW * CHUNK)

    @pl.loop(0, chunks_per_w)
    def _chunk(g):
        rb = (wid * chunks_per_w + g) * SUB
        pltpu.sync_copy(x_hbm.at[pl.ds(rb, SUB), :], idx_v)
        copies = [
            pltpu.async_copy(
                table_hbm.at[idx_v.at[j]],
                rows_v.at[pl.ds(j * IDXW, IDXW), :],
                gsem,
            )
            for j in range(SUB)
        ]
        for c in copies:
            c.wait()

        @pl.loop(0, CHUNK)
        def _scale(i):
            for j in range(D // LANES):
                sl = pl.ds(j * LANES, LANES)
                rows_v[i, sl] = rows_v[i, sl] * SCALE

        pltpu.sync_copy(rows_v, out_hbm.at[pl.ds(rb * IDXW, CHUNK), :])


_emb = functools.partial(
    pl.kernel,
    out_type=jax.ShapeDtypeStruct((B, D), jnp.float32),
    mesh=plsc.VectorSubcoreMesh(core_axis_name="c", subcore_axis_name="s"),
    scratch_types=[
        pltpu.VMEM((SUB, IDXW), jnp.int32),
        pltpu.VMEM((CHUNK, D), jnp.float32),
        pltpu.SemaphoreType.DMA,
    ],
    compiler_params=pltpu.CompilerParams(use_tc_tiling_on_sc=False),
)(_emb_body)


def kernel(x, table):
    xf = x.reshape(B // IDXW, IDXW)
    out = _emb(xf, table)
    return out.reshape(x.shape + (D,))


# speedup vs baseline: 1.1382x; 1.1382x over previous
import functools

import jax
import jax.numpy as jnp
from jax import lax
from jax.experimental import pallas as pl
from jax.experimental.pallas import tpu as pltpu
from jax.experimental.pallas import tpu_sc as plsc

D = 64
SCALE = 8.0
NC, NS = 2, 16
NW = NC * NS
IDXW = 128
SUB = 4
CHUNK = IDXW * SUB
B = 4096 * 200
LANES = 16
NCHUNK = B // (NW * CHUNK)
NPAIR = NCHUNK // 2


def _emb_body(x_hbm, table_hbm, out_hbm,
              idx_all, rows0, rows1, gsem0, gsem1, osem0, osem1):
    wid = lax.axis_index("s") * NC + lax.axis_index("c")
    xrow0 = wid * (NCHUNK * SUB)
    pltpu.sync_copy(x_hbm.at[pl.ds(xrow0, NCHUNK * SUB), :], idx_all)

    rows = (rows0, rows1)
    gsems = (gsem0, gsem1)
    osems = (osem0, osem1)

    def fire_gather(g, b):
        for j in range(SUB):
            pltpu.async_copy(
                table_hbm.at[idx_all.at[g * SUB + j]],
                rows[b].at[pl.ds(j * IDXW, IDXW), :],
                gsems[b],
            )

    def drain_gather(b):
        pltpu.make_async_copy(
            out_hbm.at[pl.ds(0, CHUNK), :], rows[b], gsems[b]).wait()

    def scale(b):
        @pl.loop(0, CHUNK, unroll=4)
        def _scale(i):
            for j in range(D // LANES):
                sl = pl.ds(j * LANES, LANES)
                rows[b][i, sl] = rows[b][i, sl] * SCALE

    def fire_scatter(g, b):
        pltpu.async_copy(
            rows[b],
            out_hbm.at[pl.ds((xrow0 + g * SUB) * IDXW, CHUNK), :],
            osems[b],
        )

    def drain_scatter(b):
        pltpu.make_async_copy(
            out_hbm.at[pl.ds(0, CHUNK), :], rows[b], osems[b]).wait()

    fire_gather(0, 0)

    @pl.loop(0, NPAIR)
    def _pair(p):
        c0 = 2 * p

        @pl.when(p > 0)
        def _():
            drain_scatter(1)
        fire_gather(c0 + 1, 1)
        drain_gather(0)
        scale(0)
        fire_scatter(c0, 0)

        @pl.when(p + 1 < NPAIR)
        def _():
            drain_scatter(0)
            fire_gather(c0 + 2, 0)
        drain_gather(1)
        scale(1)
        fire_scatter(c0 + 1, 1)

    drain_scatter(0)
    drain_scatter(1)


_emb = functools.partial(
    pl.kernel,
    out_type=jax.ShapeDtypeStruct((B, D), jnp.float32),
    mesh=plsc.VectorSubcoreMesh(core_axis_name="c", subcore_axis_name="s"),
    scratch_types=[
        pltpu.VMEM((NCHUNK * SUB, IDXW), jnp.int32),
        pltpu.VMEM((CHUNK, D), jnp.float32),
        pltpu.VMEM((CHUNK, D), jnp.float32),
        pltpu.SemaphoreType.DMA,
        pltpu.SemaphoreType.DMA,
        pltpu.SemaphoreType.DMA,
        pltpu.SemaphoreType.DMA,
    ],
    compiler_params=pltpu.CompilerParams(use_tc_tiling_on_sc=False),
)(_emb_body)


def kernel(x, table):
    xf = x.reshape(B // IDXW, IDXW)
    out = _emb(xf, table)
    return out.reshape(x.shape + (D,))
